# baseline (speedup 1.0000x reference)
import functools

import jax
import jax.numpy as jnp
from jax import lax
from jax.experimental import pallas as pl
from jax.experimental.pallas import tpu as pltpu
from jax.experimental.pallas import tpu_sc as plsc

_B = 16384
_K = 64
_SPT = _B // 16
_CI = 65536
_TPCH = 512
_ITILES = 7812
_UTILES = 781
_ITAIL0 = _ITILES * 128
_UTAIL0 = _UTILES * 128
_LCAP = 2048
_NCID = 17


def _sc_partials(sU, sI, sJ, betaI, gUT, gIT, tU2, tI2):
    mesh = plsc.VectorSubcoreMesh(core_axis_name="c", subcore_axis_name="s")

    @functools.partial(
        pl.kernel,
        out_type=jax.ShapeDtypeStruct((2 * _B,), jnp.float32),
        mesh=mesh,
        compiler_params=pltpu.CompilerParams(needs_layout_passes=False),
        scratch_types=[
            pltpu.VMEM((_SPT + 16,), jnp.int32),
            pltpu.VMEM((_SPT + 16,), jnp.int32),
            pltpu.VMEM((_SPT + 16,), jnp.int32),
            pltpu.VMEM((_SPT,), jnp.float32),
            pltpu.VMEM((_NCID * _LCAP + 128,), jnp.int32),
            pltpu.VMEM((_NCID * 16,), jnp.int32),
            pltpu.VMEM((1024,), jnp.int32),
            pltpu.VMEM((1024,), jnp.int32),
            pltpu.VMEM((1024,), jnp.float32),
            pltpu.VMEM((1024,), jnp.float32),
            pltpu.VMEM((128,), jnp.int32),
            pltpu.VMEM((128,), jnp.float32),
            pltpu.VMEM((128,), jnp.int32),
            pltpu.VMEM((128,), jnp.float32),
            pltpu.VMEM((16, 128), jnp.float32),
            pltpu.VMEM_SHARED((_UTILES * 1024,), jnp.float32),
            pltpu.VMEM_SHARED((_TPCH * 1024,), jnp.float32),
            pltpu.VMEM_SHARED((4096,), jnp.float32),
            pltpu.SemaphoreType.DMA,
            pltpu.SemaphoreType.DMA,
        ],
    )
    def body(sU_h, sI_h, sJ_h, betaI_h, gUT_h, gIT_h, tU_h, tI_h, out_h,
             idx_u, idx_i, idx_j, acc, lst, cnts,
             gwb2, uwb2, gvals, uvals, sq, sgq, utr, utm,
             tUv, gusl, chk0, tIv_sh, sem, sem2):
        core = lax.axis_index("c")
        sid = lax.axis_index("s")
        tb = sid * _SPT
        lane = lax.broadcasted_iota(jnp.int32, (16,), 0)
        bi = gvals
        bj = uvals

        pltpu.sync_copy(sU_h.at[pl.ds(tb, _SPT)], idx_u.at[pl.ds(0, _SPT)])
        pltpu.sync_copy(sI_h.at[pl.ds(tb, _SPT)], idx_i.at[pl.ds(0, _SPT)])
        pltpu.sync_copy(sJ_h.at[pl.ds(tb, _SPT)], idx_j.at[pl.ds(0, _SPT)])
        pltpu.sync_copy(tU_h, tUv)

        @pl.when(sid == 0)
        def _stage_tail_items():
            for r in range(32):
                pltpu.sync_copy(tI_h.at[r], tIv_sh.at[pl.ds(r * 128, 128)])

        def zf(g, c2):
            acc[pl.ds(g * 16, 16)] = jnp.zeros((16,), jnp.float32)
            return c2

        lax.fori_loop(0, _SPT // 16, zf, 0)

        @pl.when(core == 0)
        def _beta():
            for c in range(_SPT // 128):
                sl = pl.ds(c * 128, 128)
                pltpu.async_copy(betaI_h.at[idx_i.at[sl]], bi.at[sl], sem2)
                pltpu.async_copy(betaI_h.at[idx_j.at[sl]], bj.at[sl], sem2)
            pltpu.make_async_copy(betaI_h.at[pl.ds(0, _SPT)], bi, sem2).wait()
            pltpu.make_async_copy(betaI_h.at[pl.ds(0, _SPT)], bj, sem2).wait()

            def bf(g, c2):
                sl = pl.ds(g * 16, 16)
                acc[sl] = bi[sl] - bj[sl]
                return c2

            lax.fori_loop(0, _SPT // 16, bf, 0)

        for cid in range(_NCID):
            def mk_scan(src_ref, tag, cid=cid):
                def scan(g, cnt):
                    iv = src_ref[pl.ds(g * 16, 16)]
                    if cid == 16:
                        m = iv >= _ITAIL0
                    elif cid == 15:
                        m = ((iv >> 16) == 15) & (iv < _ITAIL0)
                    else:
                        m = (iv >> 16) == cid
                    mi = m.astype(jnp.int32)
                    pos = cnt + plsc.cumsum(mi) - mi
                    plsc.store_scatter(
                        lst, [cid * _LCAP + pos], g * 16 + lane + tag, mask=m)
                    return cnt + jnp.sum(mi)
                return scan

            cnt = lax.fori_loop(0, _SPT // 16, mk_scan(idx_i, 0), 0)
            cnt = lax.fori_loop(0, _SPT // 16, mk_scan(idx_j, _SPT), cnt)
            cnts[pl.ds(cid * 16, 16)] = jnp.full((16,), cnt, jnp.int32)

        def stage_slab(a):
            lo = jnp.minimum(sid * 49, _UTILES)
            hi = jnp.minimum((sid + 1) * 49, _UTILES)
            for r in range(8):
                row = gUT_h.at[a * 8 + r]

                def tl(t, c2, row=row, r=r):
                    pltpu.async_copy(
                        row.at[pl.ds(pl.multiple_of(t * 128, 128), 128)],
                        gusl.at[pl.ds(pl.multiple_of(t * 1024 + r * 128, 128),
                                      128)], sem)
                    return c2

                lax.fori_loop(lo, hi, tl, 0)
            for r in range(8):
                row = gUT_h.at[a * 8 + r]

                def tld(t, c2, row=row, r=r):
                    pltpu.make_async_copy(
                        row.at[pl.ds(pl.multiple_of(t * 128, 128), 128)],
                        gusl.at[pl.ds(pl.multiple_of(t * 1024 + r * 128, 128),
                                      128)], sem).wait()
                    return c2

                lax.fori_loop(lo, hi, tld, 0)

        def chunk_copies(a, cidg, buf, drain):
            ct = jnp.where(cidg == 15, _ITILES - 15 * _TPCH, _TPCH)
            tb0 = cidg * _TPCH
            lo = jnp.minimum(sid * 32, ct)
            hi = jnp.minimum((sid + 1) * 32, ct)
            for r in range(8):
                row = gIT_h.at[a * 8 + r]

                def tl(t, c2, row=row, r=r):
                    src = row.at[pl.ds(
                        pl.multiple_of((tb0 + t) * 128, 128), 128)]
                    dst = buf.at[pl.ds(
                        pl.multiple_of(t * 1024 + r * 128, 128), 128)]
                    if drain:
                        pltpu.make_async_copy(src, dst, sem).wait()
                    else:
                        pltpu.async_copy(src, dst, sem)
                    return c2

                lax.fori_loop(lo, hi, tl, 0)

        def process(a, cidsel, buf, tail_items):
            cnt = cnts[pl.ds(cidsel * 16, 16)][0]
            nq = (cnt + 127) >> 7
            if tail_items:
                nq = jnp.where(core == 1, nq, 0)

            def quantum(q, c2):
                for sg in range(8):
                    sl = pl.ds(sg * 16, 16)
                    ent = lst[pl.ds(cidsel * _LCAP + q * 128 + sg * 16, 16)]
                    valid = (q * 128 + sg * 16 + lane) < cnt
                    s = ent & (_SPT - 1)
                    isj = ent >= _SPT
                    ivi = plsc.load_gather(idx_i, [s])
                    ivj = plsc.load_gather(idx_j, [s])
                    iv = jnp.where(isj, ivj, ivi)
                    uv = plsc.load_gather(idx_u, [s])
                    sq[sl] = s
                    vf = jnp.where(valid, 1.0, 0.0).astype(jnp.float32)
                    sgq[sl] = jnp.where(isj, -vf, vf)
                    utf = uv >= _UTAIL0
                    utm[sl] = jnp.where(utf, 1.0, 0.0).astype(jnp.float32)
                    utr[sl] = jnp.where(utf, uv - _UTAIL0, 0)
                    uvc = jnp.minimum(uv, _UTAIL0 - 1)
                    uw = ((uvc >> 7) << 10) + (uvc & 127)
                    if tail_items:
                        gw = jnp.where(valid, iv - _ITAIL0, 0) * _K
                    else:
                        rel = iv & (_CI - 1)
                        gw = ((rel >> 7) << 10) + (rel & 127)
                    for r in range(8):
                        rsl = pl.ds(r * 128 + sg * 16, 16)
                        uwb2[rsl] = uw + r * 128
                        gwb2[rsl] = gw + (a * 8 + r if tail_items else r * 128)
                cps = []
                for r in range(8):
                    rsl = pl.ds(r * 128, 128)
                    cps.append(pltpu.async_copy(
                        gusl.at[uwb2.at[rsl]], uvals.at[rsl], sem2))
                    cps.append(pltpu.async_copy(
                        buf.at[gwb2.at[rsl]], gvals.at[rsl], sem2))
                for cp in cps:
                    cp.wait()
                for sg in range(8):
                    sl = pl.ds(sg * 16, 16)
                    s16 = sq[sl]
                    sg16 = sgq[sl]
                    um16 = utm[sl]
                    ur16 = utr[sl]
                    tot = jnp.zeros((16,), jnp.float32)
                    for r in range(8):
                        d = a * 8 + r
                        uvv = uvals[pl.ds(r * 128 + sg * 16, 16)]
                        uf = ur16 * _K + d
                        utv = plsc.load_gather(tUv, [uf >> 7, uf & 127])
                        uvv = uvv + um16 * (utv - uvv)
                        gv = gvals[pl.ds(r * 128 + sg * 16, 16)]
                        tot = tot + gv * uvv
                    plsc.addupdate_scatter(acc, [s16], tot * sg16)
                return c2

            lax.fori_loop(0, nq, quantum, 0)

        def a_loop(a, c2):
            stage_slab(a)
            plsc.subcore_barrier()

            def cl_loop(cl, c3):
                cidg = core * 8 + cl
                chunk_copies(a, cidg, chk0, drain=False)
                chunk_copies(a, cidg, chk0, drain=True)
                plsc.subcore_barrier()
                process(a, cidg, chk0, tail_items=False)
                plsc.subcore_barrier()
                return c3

            lax.fori_loop(0, 8, cl_loop, 0)
            process(a, 16, tIv_sh, tail_items=True)
            plsc.subcore_barrier()
            return c2

        lax.fori_loop(0, 8, a_loop, 0)

        pltpu.sync_copy(acc, out_h.at[pl.ds(core * _B + tb, _SPT)])

    return body(sU, sI, sJ, betaI, gUT, gIT, tU2, tI2)


def _tc_loss(p3):
    def bodytc(x_ref, o_ref):
        v = x_ref[0] + x_ref[1]
        sp = jnp.maximum(-v, 0.0) + jnp.log1p(jnp.exp(-jnp.abs(v)))
        o_ref[...] = (jnp.sum(sp) * (1.0 / _B)).reshape(1, 1)

    return pl.pallas_call(
        bodytc,
        out_shape=jax.ShapeDtypeStruct((1, 1), jnp.float32),
    )(p3)


def kernel(sampleU, sampleI, sampleJ, betaI, gammaU, gammaI):
    p = _sc_partials(
        sampleU, sampleI, sampleJ, betaI, gammaU.T, gammaI.T,
        gammaU[_UTAIL0:].reshape(16, 128), gammaI[_ITAIL0:].reshape(32, 128))
    return _tc_loss(p.reshape(2, 128, 128))[0, 0]


# speedup vs baseline: 3.2990x; 3.2990x over previous
import functools

import jax
import jax.numpy as jnp
from jax import lax
from jax.experimental import pallas as pl
from jax.experimental.pallas import tpu as pltpu
from jax.experimental.pallas import tpu_sc as plsc

_B = 16384
_K = 64
_SPT = _B // 16
_CI = 65536
_TPCH = 512
_ITILES = 7812
_UTILES = 781
_ITAIL0 = _ITILES * 128
_UTAIL0 = _UTILES * 128
_LCAP = 2048
_NCID = 17
_UREG = _UTILES * 128
_CREG = _CI


def _sc_partials(sU, sI, sJ, betaI, gUT, gIT, tU2, tI2):
    mesh = plsc.VectorSubcoreMesh(core_axis_name="c", subcore_axis_name="s")

    @functools.partial(
        pl.kernel,
        out_type=jax.ShapeDtypeStruct((2 * _B,), jnp.float32),
        mesh=mesh,
        compiler_params=pltpu.CompilerParams(needs_layout_passes=False),
        scratch_types=[
            pltpu.VMEM((_SPT + 16,), jnp.int32),
            pltpu.VMEM((_SPT + 16,), jnp.int32),
            pltpu.VMEM((_SPT + 16,), jnp.int32),
            pltpu.VMEM((_SPT,), jnp.float32),
            pltpu.VMEM((_NCID * _LCAP + 128,), jnp.int32),
            pltpu.VMEM((_NCID * 16,), jnp.int32),
            pltpu.VMEM((1024,), jnp.int32),
            pltpu.VMEM((1024,), jnp.int32),
            pltpu.VMEM((1024,), jnp.float32),
            pltpu.VMEM((1024,), jnp.float32),
            pltpu.VMEM((128,), jnp.int32),
            pltpu.VMEM((128,), jnp.float32),
            pltpu.VMEM((128,), jnp.int32),
            pltpu.VMEM((128,), jnp.float32),
            pltpu.VMEM((16, 128), jnp.float32),
            pltpu.VMEM_SHARED((_UTILES * 1024,), jnp.float32),
            pltpu.VMEM_SHARED((_TPCH * 1024,), jnp.float32),
            pltpu.VMEM_SHARED((4096,), jnp.float32),
            pltpu.SemaphoreType.DMA,
            pltpu.SemaphoreType.DMA,
        ],
    )
    def body(sU_h, sI_h, sJ_h, betaI_h, gUT_h, gIT_h, tU_h, tI_h, out_h,
             idx_u, idx_i, idx_j, acc, lst, cnts,
             gwb2, uwb2, gvals, uvals, sq, sgq, utr, utm,
             tUv, gusl, chk0, tIv_sh, sem, sem2):
        core = lax.axis_index("c")
        sid = lax.axis_index("s")
        tb = sid * _SPT
        lane = lax.broadcasted_iota(jnp.int32, (16,), 0)
        bi = gvals
        bj = uvals

        pltpu.sync_copy(sU_h.at[pl.ds(tb, _SPT)], idx_u.at[pl.ds(0, _SPT)])
        pltpu.sync_copy(sI_h.at[pl.ds(tb, _SPT)], idx_i.at[pl.ds(0, _SPT)])
        pltpu.sync_copy(sJ_h.at[pl.ds(tb, _SPT)], idx_j.at[pl.ds(0, _SPT)])
        pltpu.sync_copy(tU_h, tUv)

        @pl.when(sid == 0)
        def _stage_tail_items():
            for r in range(32):
                pltpu.sync_copy(tI_h.at[r], tIv_sh.at[pl.ds(r * 128, 128)])

        def zf(g, c2):
            acc[pl.ds(g * 16, 16)] = jnp.zeros((16,), jnp.float32)
            return c2

        lax.fori_loop(0, _SPT // 16, zf, 0)

        @pl.when(core == 0)
        def _beta():
            for c in range(_SPT // 128):
                sl = pl.ds(c * 128, 128)
                pltpu.async_copy(betaI_h.at[idx_i.at[sl]], bi.at[sl], sem2)
                pltpu.async_copy(betaI_h.at[idx_j.at[sl]], bj.at[sl], sem2)
            pltpu.make_async_copy(betaI_h.at[pl.ds(0, _SPT)], bi, sem2).wait()
            pltpu.make_async_copy(betaI_h.at[pl.ds(0, _SPT)], bj, sem2).wait()

            def bf(g, c2):
                sl = pl.ds(g * 16, 16)
                acc[sl] = bi[sl] - bj[sl]
                return c2

            lax.fori_loop(0, _SPT // 16, bf, 0)

        for cid in range(_NCID):
            def mk_scan(src_ref, tag, cid=cid):
                def scan(g, cnt):
                    iv = src_ref[pl.ds(g * 16, 16)]
                    if cid == 16:
                        m = iv >= _ITAIL0
                    elif cid == 15:
                        m = ((iv >> 16) == 15) & (iv < _ITAIL0)
                    else:
                        m = (iv >> 16) == cid
                    mi = m.astype(jnp.int32)
                    pos = cnt + plsc.cumsum(mi) - mi
                    plsc.store_scatter(
                        lst, [cid * _LCAP + pos], g * 16 + lane + tag, mask=m)
                    return cnt + jnp.sum(mi)
                return scan

            cnt = lax.fori_loop(0, _SPT // 16, mk_scan(idx_i, 0), 0)
            cnt = lax.fori_loop(0, _SPT // 16, mk_scan(idx_j, _SPT), cnt)
            cnts[pl.ds(cid * 16, 16)] = jnp.full((16,), cnt, jnp.int32)

        def stage_slab(a):
            rows = [gUT_h.at[a * 8 + r] for r in range(8)]
            lo = jnp.minimum(sid * 7, 97)
            hi = jnp.minimum((sid + 1) * 7, 97)

            def mk(drain):
                def tl(st, c2):
                    for r in range(8):
                        src = rows[r].at[pl.ds(
                            pl.multiple_of(st * 1024, 128), 1024)]
                        dst = gusl.at[pl.ds(
                            pl.multiple_of(r * _UREG + st * 1024, 128), 1024)]
                        if drain:
                            pltpu.make_async_copy(src, dst, sem).wait()
                        else:
                            pltpu.async_copy(src, dst, sem)
                    return c2
                return tl

            lax.fori_loop(lo, hi, mk(False), 0)

            @pl.when(sid == 15)
            def _slab_tail():
                for t in range(97 * 8, _UTILES):
                    for r in range(8):
                        pltpu.async_copy(
                            rows[r].at[pl.ds(t * 128, 128)],
                            gusl.at[pl.ds(r * _UREG + t * 128, 128)], sem)

            lax.fori_loop(lo, hi, mk(True), 0)

            @pl.when(sid == 15)
            def _slab_tail_drain():
                for t in range(97 * 8, _UTILES):
                    for r in range(8):
                        pltpu.make_async_copy(
                            rows[r].at[pl.ds(t * 128, 128)],
                            gusl.at[pl.ds(r * _UREG + t * 128, 128)],
                            sem).wait()

        def chunk_copies(a, cidg, buf, drain):
            rows = [gIT_h.at[a * 8 + r] for r in range(8)]
            sct = jnp.where(cidg == 15, 16, 64)
            wb0 = cidg * _TPCH * 128
            lo = jnp.minimum(sid * 4, sct)
            hi = jnp.minimum((sid + 1) * 4, sct)

            def tl(st, c2):
                for r in range(8):
                    src = rows[r].at[pl.ds(
                        pl.multiple_of(wb0 + st * 1024, 128), 1024)]
                    dst = buf.at[pl.ds(
                        pl.multiple_of(r * _CREG + st * 1024, 128), 1024)]
                    if drain:
                        pltpu.make_async_copy(src, dst, sem).wait()
                    else:
                        pltpu.async_copy(src, dst, sem)
                return c2

            lax.fori_loop(lo, hi, tl, 0)

            @pl.when((sid == 15) & (cidg == 15))
            def _chunk_tail():
                for t in range(128, 132):
                    for r in range(8):
                        src = rows[r].at[pl.ds(
                            pl.multiple_of(wb0 + t * 128, 128), 128)]
                        dst = buf.at[pl.ds(
                            pl.multiple_of(r * _CREG + t * 128, 128), 128)]
                        if drain:
                            pltpu.make_async_copy(src, dst, sem).wait()
                        else:
                            pltpu.async_copy(src, dst, sem)

        def process(a, cidsel, buf, tail_items):
            cnt = cnts[pl.ds(cidsel * 16, 16)][0]
            nq = (cnt + 127) >> 7
            if tail_items:
                nq = jnp.where(core == 1, nq, 0)

            def quantum(q, c2):
                for sg in range(8):
                    sl = pl.ds(sg * 16, 16)
                    ent = lst[pl.ds(cidsel * _LCAP + q * 128 + sg * 16, 16)]
                    valid = (q * 128 + sg * 16 + lane) < cnt
                    s = ent & (_SPT - 1)
                    isj = ent >= _SPT
                    ivi = plsc.load_gather(idx_i, [s])
                    ivj = plsc.load_gather(idx_j, [s])
                    iv = jnp.where(isj, ivj, ivi)
                    uv = plsc.load_gather(idx_u, [s])
                    sq[sl] = s
                    vf = jnp.where(valid, 1.0, 0.0).astype(jnp.float32)
                    sgq[sl] = jnp.where(isj, -vf, vf)
                    utf = uv >= _UTAIL0
                    utm[sl] = jnp.where(utf, 1.0, 0.0).astype(jnp.float32)
                    utr[sl] = jnp.where(utf, uv - _UTAIL0, 0)
                    uw = jnp.minimum(uv, _UTAIL0 - 1)
                    if tail_items:
                        gw = jnp.where(valid, iv - _ITAIL0, 0) * _K
                    else:
                        gw = iv & (_CI - 1)
                    for r in range(8):
                        rsl = pl.ds(r * 128 + sg * 16, 16)
                        uwb2[rsl] = uw + r * _UREG
                        gwb2[rsl] = gw + (a * 8 + r if tail_items
                                          else r * _CREG)
                cps = []
                for r in range(8):
                    rsl = pl.ds(r * 128, 128)
                    cps.append(pltpu.async_copy(
                        gusl.at[uwb2.at[rsl]], uvals.at[rsl], sem2))
                    cps.append(pltpu.async_copy(
                        buf.at[gwb2.at[rsl]], gvals.at[rsl], sem2))
                for cp in cps:
                    cp.wait()
                for sg in range(8):
                    sl = pl.ds(sg * 16, 16)
                    s16 = sq[sl]
                    sg16 = sgq[sl]
                    um16 = utm[sl]
                    ur16 = utr[sl]
                    tot = jnp.zeros((16,), jnp.float32)
                    for r in range(8):
                        d = a * 8 + r
                        uvv = uvals[pl.ds(r * 128 + sg * 16, 16)]
                        uf = ur16 * _K + d
                        utv = plsc.load_gather(tUv, [uf >> 7, uf & 127])
                        uvv = uvv + um16 * (utv - uvv)
                        gv = gvals[pl.ds(r * 128 + sg * 16, 16)]
                        tot = tot + gv * uvv
                    plsc.addupdate_scatter(acc, [s16], tot * sg16)
                return c2

            lax.fori_loop(0, nq, quantum, 0)

        def a_loop(a, c2):
            stage_slab(a)
            plsc.subcore_barrier()

            def cl_loop(cl, c3):
                cidg = core * 8 + cl
                chunk_copies(a, cidg, chk0, drain=False)
                chunk_copies(a, cidg, chk0, drain=True)
                plsc.subcore_barrier()
                process(a, cidg, chk0, tail_items=False)
                plsc.subcore_barrier()
                return c3

            lax.fori_loop(0, 8, cl_loop, 0)
            process(a, 16, tIv_sh, tail_items=True)
            plsc.subcore_barrier()
            return c2

        lax.fori_loop(0, 8, a_loop, 0)

        pltpu.sync_copy(acc, out_h.at[pl.ds(core * _B + tb, _SPT)])

    return body(sU, sI, sJ, betaI, gUT, gIT, tU2, tI2)


def _tc_loss(p3):
    def bodytc(x_ref, o_ref):
        v = x_ref[0] + x_ref[1]
        sp = jnp.maximum(-v, 0.0) + jnp.log1p(jnp.exp(-jnp.abs(v)))
        o_ref[...] = (jnp.sum(sp) * (1.0 / _B)).reshape(1, 1)

    return pl.pallas_call(
        bodytc,
        out_shape=jax.ShapeDtypeStruct((1, 1), jnp.float32),
    )(p3)


def kernel(sampleU, sampleI, sampleJ, betaI, gammaU, gammaI):
    p = _sc_partials(
        sampleU, sampleI, sampleJ, betaI, gammaU.T, gammaI.T,
        gammaU[_UTAIL0:].reshape(16, 128), gammaI[_ITAIL0:].reshape(32, 128))
    return _tc_loss(p.reshape(2, 128, 128))[0, 0]
